# baseline (speedup 1.0000x reference)
import functools

import jax
import jax.numpy as jnp
import numpy as np
from jax import lax
from jax.experimental import pallas as pl
from jax.experimental.pallas import tpu as pltpu
from jax.experimental.pallas import tpu_sc as plsc

N = 10000
E = 320000
D = 128
H = 128
C = 40

N_PAD = 10240
NW = 32
E_PAD = 327680
CB = 128
CHUNKS = E_PAD // NW // CB
CH = CHUNKS // 2
ROWS_PER_TILE = N_PAD // 16
DW = 16

_INV_BN = float(1.0 / np.sqrt(1.0 + 1e-5))


def _deg_body(src3, dst3, ones_hbm, zeros_hbm, d_src, d_dst,
              sidx_v, didx_v, ones_v, deg_l, row_v, sem, acc_s, acc_d):
    cid = lax.axis_index("c")
    sid = lax.axis_index("s")
    wid = cid * 16 + sid
    base = sid * ROWS_PER_TILE

    pltpu.sync_copy(ones_hbm, ones_v)
    with jax.named_scope("deg_zero"):
        pltpu.sync_copy(zeros_hbm.at[pl.ds(base, ROWS_PER_TILE)],
                        acc_s.at[pl.ds(base, ROWS_PER_TILE)])
        pltpu.sync_copy(zeros_hbm.at[pl.ds(base, ROWS_PER_TILE)],
                        acc_d.at[pl.ds(base, ROWS_PER_TILE)])
        plsc.subcore_barrier()

    with jax.named_scope("deg_loop"):
        for half in range(2):
            pltpu.sync_copy(src3.at[wid, pl.ds(half * CH, CH)], sidx_v)
            pltpu.sync_copy(dst3.at[wid, pl.ds(half * CH, CH)], didx_v)

            @pl.loop(0, CH, step=5)
            def _(j):
                for k in range(5):
                    pltpu.async_copy(ones_v, acc_s.at[sidx_v.at[j + k]],
                                     sem, add=True)
                    pltpu.async_copy(ones_v, acc_d.at[didx_v.at[j + k]],
                                     sem, add=True)
                for k in range(5):
                    pltpu.make_async_copy(ones_v, acc_s.at[sidx_v.at[j + k]],
                                          sem).wait()
                    pltpu.make_async_copy(ones_v, acc_d.at[didx_v.at[j + k]],
                                          sem).wait()

    plsc.subcore_barrier()

    with jax.named_scope("deg_bcast"):
        for acc, out in ((acc_s, d_src), (acc_d, d_dst)):
            pltpu.sync_copy(acc.at[pl.ds(base, ROWS_PER_TILE)], deg_l)

            @pl.loop(0, ROWS_PER_TILE // 16)
            def _(i):
                v16 = deg_l[pl.ds(i * 16, 16)]
                for lane in range(16):
                    vec = jnp.full((16,), v16[lane], dtype=jnp.float32)
                    for c in range(8):
                        row_v[i * 16 + lane, pl.ds(c * 16, 16)] = vec

            pltpu.sync_copy(row_v, out.at[cid, pl.ds(base, ROWS_PER_TILE)])


@functools.cache
def _sc_kernels():
    mesh = plsc.VectorSubcoreMesh(core_axis_name="c", subcore_axis_name="s")
    sc_degrees = pl.kernel(
        _deg_body,
        out_type=[jax.ShapeDtypeStruct((2, N_PAD, H), jnp.float32),
                  jax.ShapeDtypeStruct((2, N_PAD, H), jnp.float32)],
        mesh=mesh,
        scratch_types=[pltpu.VMEM((CH, CB), jnp.int32),
                       pltpu.VMEM((CH, CB), jnp.int32),
                       pltpu.VMEM((CB,), jnp.float32),
                       pltpu.VMEM((ROWS_PER_TILE,), jnp.float32),
                       pltpu.VMEM((ROWS_PER_TILE, H), jnp.float32),
                       pltpu.SemaphoreType.DMA,
                       pltpu.VMEM_SHARED((N_PAD,), jnp.float32),
                       pltpu.VMEM_SHARED((N_PAD,), jnp.float32)],
    )
    sc_scatter = pl.kernel(
        _scat_body,
        out_type=jax.ShapeDtypeStruct((2, N_PAD, H), jnp.float32),
        mesh=mesh,
        scratch_types=[pltpu.VMEM((CH, CB), jnp.int32),
                       pltpu.VMEM((CH, CB), jnp.int32),
                       pltpu.VMEM((CB, H), jnp.float32),
                       pltpu.VMEM((CB, H), jnp.float32),
                       pltpu.SemaphoreType.DMA,
                       pltpu.SemaphoreType.DMA,
                       pltpu.SemaphoreType.DMA,
                       pltpu.SemaphoreType.DMA,
                       pltpu.VMEM_SHARED((N_PAD, H), jnp.float32)],
    )
    return sc_degrees, sc_scatter


def _scat_body(hw, src3, dst3, zeros_hbm, out,
               sidx_v, didx_v, rows_a, rows_b,
               g_sem_a, g_sem_b, s_sem_a, s_sem_b, acc_sh):
    cid = lax.axis_index("c")
    sid = lax.axis_index("s")
    wid = cid * 16 + sid
    base = sid * ROWS_PER_TILE

    def g_start(j, buf, sem):
        pltpu.async_copy(hw.at[sidx_v.at[j]], buf, sem)

    def g_wait(j, buf, sem):
        pltpu.make_async_copy(hw.at[sidx_v.at[j]], buf, sem).wait()

    def s_start(j, buf, sem):
        pltpu.async_copy(buf, acc_sh.at[didx_v.at[j]], sem, add=True)

    def s_wait(j, buf, sem):
        pltpu.make_async_copy(buf, acc_sh.at[didx_v.at[j]], sem).wait()

    with jax.named_scope("stage_idx"):
        pltpu.sync_copy(src3.at[wid, pl.ds(0, CH)], sidx_v)
        pltpu.sync_copy(dst3.at[wid, pl.ds(0, CH)], didx_v)
    g_start(0, rows_a, g_sem_a)
    with jax.named_scope("zero_acc"):
        pltpu.sync_copy(zeros_hbm.at[pl.ds(base, ROWS_PER_TILE)],
                        acc_sh.at[pl.ds(base, ROWS_PER_TILE)])
        plsc.subcore_barrier()

    for half in range(2):
        if half:
            with jax.named_scope("stage_idx"):
                pltpu.sync_copy(src3.at[wid, pl.ds(half * CH, CH)], sidx_v)
                pltpu.sync_copy(dst3.at[wid, pl.ds(half * CH, CH)], didx_v)
            g_start(0, rows_a, g_sem_a)

        with jax.named_scope("edge_loop"):
            @pl.loop(0, CH, step=2)
            def _(j):
                g_start(j + 1, rows_b, g_sem_b)
                g_wait(j, rows_a, g_sem_a)
                s_start(j, rows_a, s_sem_a)
                g_wait(j + 1, rows_b, g_sem_b)
                s_start(j + 1, rows_b, s_sem_b)
                s_wait(j, rows_a, s_sem_a)

                @pl.when(j + 2 < CH)
                def _():
                    g_start(j + 2, rows_a, g_sem_a)

                s_wait(j + 1, rows_b, s_sem_b)

    with jax.named_scope("writeout"):
        plsc.subcore_barrier()
        pltpu.sync_copy(acc_sh.at[pl.ds(base, ROWS_PER_TILE)],
                        out.at[cid, pl.ds(base, ROWS_PER_TILE)])


_BLK = 1024
_GRID = N_PAD // _BLK

_row_spec = pl.BlockSpec((_BLK, H), lambda i: (i, 0))
_pair_spec = pl.BlockSpec((2, _BLK, H), lambda i: (0, i, 0))
_deg_spec = pl.BlockSpec((2, _BLK, H), lambda i: (0, i, 0))
_w_spec = pl.BlockSpec((H, H), lambda i: (0, 0))
_vec_spec = pl.BlockSpec((1, H), lambda i: (0, 0))


def _pre_body(d_src, d_dst, feat, w0, r_out, r_in, hw0):
    ds = d_src[0, :, 0:1] + d_src[1, :, 0:1]
    dd = d_dst[0, :, 0:1] + d_dst[1, :, 0:1]
    ro = jnp.broadcast_to(lax.rsqrt(jnp.maximum(ds, 1.0)), (_BLK, H))
    ri = jnp.broadcast_to(lax.rsqrt(jnp.maximum(dd, 1.0)), (_BLK, H))
    r_out[...] = ro
    r_in[...] = ri
    hw0[...] = jnp.dot(feat[...] * ro, w0[...],
                       preferred_element_type=jnp.float32)


_tc_pre = pl.pallas_call(
    _pre_body,
    grid=(_GRID,),
    in_specs=[_deg_spec, _deg_spec, _row_spec, _w_spec],
    out_specs=[_row_spec, _row_spec, _row_spec],
    out_shape=[jax.ShapeDtypeStruct((N_PAD, H), jnp.float32)] * 3,
)


def _layer_body(a, r_in, r_out, g, b, beta, w, h_out, hw_out):
    gc = g[...] * _INV_BN
    bc = b[...] * gc + beta[...]
    agg = a[0] + a[1]
    h = jnp.maximum(agg * r_in[...] * gc + bc, 0.0)
    h_out[...] = h
    hw_out[...] = jnp.dot(h * r_out[...], w[...],
                          preferred_element_type=jnp.float32)


_tc_layer = pl.pallas_call(
    _layer_body,
    grid=(_GRID,),
    in_specs=[_pair_spec, _row_spec, _row_spec,
              _vec_spec, _vec_spec, _vec_spec, _w_spec],
    out_specs=[_row_spec, _row_spec],
    out_shape=[jax.ShapeDtypeStruct((N_PAD, H), jnp.float32)] * 2,
)


def _final_body(a, r_in, g, b, beta, h1, h2, wl, blin, out):
    gc = g[...] * _INV_BN
    bc = b[...] * gc + beta[...]
    agg = a[0] + a[1]
    h3 = jnp.maximum(agg * r_in[...] * gc + bc, 0.0)
    acc = jnp.dot(h1[...], wl[0], preferred_element_type=jnp.float32)
    acc += jnp.dot(h2[...], wl[1], preferred_element_type=jnp.float32)
    acc += jnp.dot(h3, wl[2], preferred_element_type=jnp.float32)
    out[...] = acc + blin[...]


_tc_final = pl.pallas_call(
    _final_body,
    grid=(_GRID,),
    in_specs=[_pair_spec, _row_spec,
              _vec_spec, _vec_spec, _vec_spec,
              _row_spec, _row_spec,
              pl.BlockSpec((3, H, C), lambda i: (0, 0, 0)),
              pl.BlockSpec((1, C), lambda i: (0, 0))],
    out_specs=pl.BlockSpec((_BLK, C), lambda i: (i, 0)),
    out_shape=jax.ShapeDtypeStruct((N, C), jnp.float32),
)


def kernel(feat, edge_index, W0, W1, W2, b0, b1, b2, g0, g1, g2,
           beta0, beta1, beta2, W_lin, b_lin):
    f32 = jnp.float32
    feat_p = jnp.zeros((N_PAD, D), f32).at[:N].set(feat)

    pad = N + (jnp.arange(E_PAD - E, dtype=jnp.int32) % (N_PAD - N))
    src3 = jnp.concatenate([edge_index[0], pad]).reshape(NW, CHUNKS, CB)
    dst3 = jnp.concatenate([edge_index[1], pad]).reshape(NW, CHUNKS, CB)

    ones_r = jnp.ones((CB,), f32)
    zeros_1d = jnp.zeros((N_PAD,), f32)
    zeros_big = jnp.zeros((N_PAD, H), f32)

    sc_degrees, sc_scatter = _sc_kernels()
    d_src, d_dst = sc_degrees(src3, dst3, ones_r, zeros_1d)
    r_out, r_in, hw0 = _tc_pre(d_src, d_dst, feat_p, W0)

    a1 = sc_scatter(hw0, src3, dst3, zeros_big)
    h1, hw1 = _tc_layer(a1, r_in, r_out,
                        g0.reshape(1, H), b0.reshape(1, H),
                        beta0.reshape(1, H), W1)

    a2 = sc_scatter(hw1, src3, dst3, zeros_big)
    h2, hw2 = _tc_layer(a2, r_in, r_out,
                        g1.reshape(1, H), b1.reshape(1, H),
                        beta1.reshape(1, H), W2)

    a3 = sc_scatter(hw2, src3, dst3, zeros_big)

    return _tc_final(a3, r_in,
                     g2.reshape(1, H), b2.reshape(1, H),
                     beta2.reshape(1, H), h1, h2,
                     W_lin.reshape(3, H, C), b_lin.reshape(1, C))


# speedup vs baseline: 1.2238x; 1.2238x over previous
import functools

import jax
import jax.numpy as jnp
import numpy as np
from jax import lax
from jax.experimental import pallas as pl
from jax.experimental.pallas import tpu as pltpu
from jax.experimental.pallas import tpu_sc as plsc

N = 10000
E = 320000
D = 128
H = 128
C = 40

N_PAD = 10240
NW = 32
E_PAD = 327680
CB = 128
CHUNKS = E_PAD // NW // CB
CH = CHUNKS // 2
ROWS_PER_TILE = N_PAD // 16
DW = 16

_INV_BN = float(1.0 / np.sqrt(1.0 + 1e-5))


def _deg_body(src3, dst3, ones_hbm, zeros_hbm, d_src, d_dst,
              sidx_v, didx_v, ones_v, deg_l, row_v, sem, acc_s, acc_d):
    cid = lax.axis_index("c")
    sid = lax.axis_index("s")
    wid = cid * 16 + sid
    base = sid * ROWS_PER_TILE

    pltpu.sync_copy(ones_hbm, ones_v)
    with jax.named_scope("deg_zero"):
        pltpu.sync_copy(zeros_hbm.at[pl.ds(base, ROWS_PER_TILE)],
                        acc_s.at[pl.ds(base, ROWS_PER_TILE)])
        pltpu.sync_copy(zeros_hbm.at[pl.ds(base, ROWS_PER_TILE)],
                        acc_d.at[pl.ds(base, ROWS_PER_TILE)])
        plsc.subcore_barrier()

    with jax.named_scope("deg_loop"):
        for half in range(2):
            pltpu.sync_copy(src3.at[wid, pl.ds(half * CH, CH)], sidx_v)
            pltpu.sync_copy(dst3.at[wid, pl.ds(half * CH, CH)], didx_v)

            @pl.loop(0, CH, step=5)
            def _(j):
                for k in range(5):
                    pltpu.async_copy(ones_v, acc_s.at[sidx_v.at[j + k]],
                                     sem, add=True)
                    pltpu.async_copy(ones_v, acc_d.at[didx_v.at[j + k]],
                                     sem, add=True)
                for k in range(5):
                    pltpu.make_async_copy(ones_v, acc_s.at[sidx_v.at[j + k]],
                                          sem).wait()
                    pltpu.make_async_copy(ones_v, acc_d.at[didx_v.at[j + k]],
                                          sem).wait()

    plsc.subcore_barrier()

    with jax.named_scope("deg_bcast"):
        for acc, out in ((acc_s, d_src), (acc_d, d_dst)):
            pltpu.sync_copy(acc.at[pl.ds(base, ROWS_PER_TILE)], deg_l)

            @pl.loop(0, ROWS_PER_TILE // 16)
            def _(i):
                v16 = deg_l[pl.ds(i * 16, 16)]
                for lane in range(16):
                    vec = jnp.full((16,), v16[lane], dtype=jnp.float32)
                    for c in range(8):
                        row_v[i * 16 + lane, pl.ds(c * 16, 16)] = vec

            pltpu.sync_copy(row_v, out.at[cid, pl.ds(base, ROWS_PER_TILE)])


@functools.cache
def _sc_kernels():
    mesh = plsc.VectorSubcoreMesh(core_axis_name="c", subcore_axis_name="s")
    sc_degrees = pl.kernel(
        _deg_body,
        out_type=[jax.ShapeDtypeStruct((2, N_PAD, H), jnp.float32),
                  jax.ShapeDtypeStruct((2, N_PAD, H), jnp.float32)],
        mesh=mesh,
        scratch_types=[pltpu.VMEM((CH, CB), jnp.int32),
                       pltpu.VMEM((CH, CB), jnp.int32),
                       pltpu.VMEM((CB,), jnp.float32),
                       pltpu.VMEM((ROWS_PER_TILE,), jnp.float32),
                       pltpu.VMEM((ROWS_PER_TILE, H), jnp.float32),
                       pltpu.SemaphoreType.DMA,
                       pltpu.VMEM_SHARED((N_PAD,), jnp.float32),
                       pltpu.VMEM_SHARED((N_PAD,), jnp.float32)],
    )
    sc_scatter = pl.kernel(
        _scat_body,
        out_type=jax.ShapeDtypeStruct((2, N_PAD, H), jnp.float32),
        mesh=mesh,
        scratch_types=[pltpu.VMEM((CH, CB), jnp.int32),
                       pltpu.VMEM((CH, CB), jnp.int32),
                       pltpu.VMEM((CB, H), jnp.float32),
                       pltpu.VMEM((CB, H), jnp.float32),
                       pltpu.SemaphoreType.DMA,
                       pltpu.SemaphoreType.DMA,
                       pltpu.VMEM_SHARED((N_PAD, H), jnp.float32)],
    )
    return sc_degrees, sc_scatter


def _scat_body(hw, src3, dst3, zeros_hbm, out,
               sidx_v, didx_v, rows_a, rows_b,
               g_sem_a, g_sem_b, acc_sh):
    cid = lax.axis_index("c")
    sid = lax.axis_index("s")
    wid = cid * 16 + sid
    base = sid * ROWS_PER_TILE

    def g_start(j, buf, sem):
        pltpu.async_copy(hw.at[sidx_v.at[j]], buf, sem)

    def g_wait(j, buf, sem):
        pltpu.make_async_copy(hw.at[sidx_v.at[j]], buf, sem).wait()

    with jax.named_scope("stage_idx"):
        pltpu.sync_copy(src3.at[wid, pl.ds(0, CH)], sidx_v)
        pltpu.sync_copy(dst3.at[wid, pl.ds(0, CH)], didx_v)
    g_start(0, rows_a, g_sem_a)
    with jax.named_scope("zero_acc"):
        pltpu.sync_copy(zeros_hbm.at[pl.ds(base, ROWS_PER_TILE)],
                        acc_sh.at[pl.ds(base, ROWS_PER_TILE)])
        plsc.subcore_barrier()

    for half in range(2):
        if half:
            with jax.named_scope("stage_idx"):
                pltpu.sync_copy(src3.at[wid, pl.ds(half * CH, CH)], sidx_v)
                pltpu.sync_copy(dst3.at[wid, pl.ds(half * CH, CH)], didx_v)
            g_start(0, rows_a, g_sem_a)

        with jax.named_scope("edge_loop"):
            @pl.loop(0, CH, step=2)
            def _(j):
                g_start(j + 1, rows_b, g_sem_b)
                g_wait(j, rows_a, g_sem_a)
                pltpu.sync_copy(rows_a, acc_sh.at[didx_v.at[j]], add=True)

                @pl.when(j + 2 < CH)
                def _():
                    g_start(j + 2, rows_a, g_sem_a)

                g_wait(j + 1, rows_b, g_sem_b)
                pltpu.sync_copy(rows_b, acc_sh.at[didx_v.at[j + 1]], add=True)

    with jax.named_scope("writeout"):
        plsc.subcore_barrier()
        pltpu.sync_copy(acc_sh.at[pl.ds(base, ROWS_PER_TILE)],
                        out.at[cid, pl.ds(base, ROWS_PER_TILE)])


_BLK = 1024
_GRID = N_PAD // _BLK

_row_spec = pl.BlockSpec((_BLK, H), lambda i: (i, 0))
_pair_spec = pl.BlockSpec((2, _BLK, H), lambda i: (0, i, 0))
_deg_spec = pl.BlockSpec((2, _BLK, H), lambda i: (0, i, 0))
_w_spec = pl.BlockSpec((H, H), lambda i: (0, 0))
_vec_spec = pl.BlockSpec((1, H), lambda i: (0, 0))


def _pre_body(d_src, d_dst, feat, w0, r_out, r_in, hw0):
    ds = d_src[0, :, 0:1] + d_src[1, :, 0:1]
    dd = d_dst[0, :, 0:1] + d_dst[1, :, 0:1]
    ro = jnp.broadcast_to(lax.rsqrt(jnp.maximum(ds, 1.0)), (_BLK, H))
    ri = jnp.broadcast_to(lax.rsqrt(jnp.maximum(dd, 1.0)), (_BLK, H))
    r_out[...] = ro
    r_in[...] = ri
    hw0[...] = jnp.dot(feat[...] * ro, w0[...],
                       preferred_element_type=jnp.float32)


_tc_pre = pl.pallas_call(
    _pre_body,
    grid=(_GRID,),
    in_specs=[_deg_spec, _deg_spec, _row_spec, _w_spec],
    out_specs=[_row_spec, _row_spec, _row_spec],
    out_shape=[jax.ShapeDtypeStruct((N_PAD, H), jnp.float32)] * 3,
)


def _layer_body(a, r_in, r_out, g, b, beta, w, h_out, hw_out):
    gc = g[...] * _INV_BN
    bc = b[...] * gc + beta[...]
    agg = a[0] + a[1]
    h = jnp.maximum(agg * r_in[...] * gc + bc, 0.0)
    h_out[...] = h
    hw_out[...] = jnp.dot(h * r_out[...], w[...],
                          preferred_element_type=jnp.float32)


_tc_layer = pl.pallas_call(
    _layer_body,
    grid=(_GRID,),
    in_specs=[_pair_spec, _row_spec, _row_spec,
              _vec_spec, _vec_spec, _vec_spec, _w_spec],
    out_specs=[_row_spec, _row_spec],
    out_shape=[jax.ShapeDtypeStruct((N_PAD, H), jnp.float32)] * 2,
)


def _final_body(a, r_in, g, b, beta, h1, h2, wl, blin, out):
    gc = g[...] * _INV_BN
    bc = b[...] * gc + beta[...]
    agg = a[0] + a[1]
    h3 = jnp.maximum(agg * r_in[...] * gc + bc, 0.0)
    acc = jnp.dot(h1[...], wl[0], preferred_element_type=jnp.float32)
    acc += jnp.dot(h2[...], wl[1], preferred_element_type=jnp.float32)
    acc += jnp.dot(h3, wl[2], preferred_element_type=jnp.float32)
    out[...] = acc + blin[...]


_tc_final = pl.pallas_call(
    _final_body,
    grid=(_GRID,),
    in_specs=[_pair_spec, _row_spec,
              _vec_spec, _vec_spec, _vec_spec,
              _row_spec, _row_spec,
              pl.BlockSpec((3, H, C), lambda i: (0, 0, 0)),
              pl.BlockSpec((1, C), lambda i: (0, 0))],
    out_specs=pl.BlockSpec((_BLK, C), lambda i: (i, 0)),
    out_shape=jax.ShapeDtypeStruct((N, C), jnp.float32),
)


def kernel(feat, edge_index, W0, W1, W2, b0, b1, b2, g0, g1, g2,
           beta0, beta1, beta2, W_lin, b_lin):
    f32 = jnp.float32
    feat_p = jnp.zeros((N_PAD, D), f32).at[:N].set(feat)

    pad = N + (jnp.arange(E_PAD - E, dtype=jnp.int32) % (N_PAD - N))
    src3 = jnp.concatenate([edge_index[0], pad]).reshape(NW, CHUNKS, CB)
    dst3 = jnp.concatenate([edge_index[1], pad]).reshape(NW, CHUNKS, CB)

    ones_r = jnp.ones((CB,), f32)
    zeros_1d = jnp.zeros((N_PAD,), f32)
    zeros_big = jnp.zeros((N_PAD, H), f32)

    sc_degrees, sc_scatter = _sc_kernels()
    d_src, d_dst = sc_degrees(src3, dst3, ones_r, zeros_1d)
    r_out, r_in, hw0 = _tc_pre(d_src, d_dst, feat_p, W0)

    a1 = sc_scatter(hw0, src3, dst3, zeros_big)
    h1, hw1 = _tc_layer(a1, r_in, r_out,
                        g0.reshape(1, H), b0.reshape(1, H),
                        beta0.reshape(1, H), W1)

    a2 = sc_scatter(hw1, src3, dst3, zeros_big)
    h2, hw2 = _tc_layer(a2, r_in, r_out,
                        g1.reshape(1, H), b1.reshape(1, H),
                        beta1.reshape(1, H), W2)

    a3 = sc_scatter(hw2, src3, dst3, zeros_big)

    return _tc_final(a3, r_in,
                     g2.reshape(1, H), b2.reshape(1, H),
                     beta2.reshape(1, H), h1, h2,
                     W_lin.reshape(3, H, C), b_lin.reshape(1, C))
